# baseline (speedup 1.0000x reference)
import functools

import jax
import jax.numpy as jnp
from jax import lax
from jax.experimental import pallas as pl
from jax.experimental.pallas import tpu as pltpu
from jax.experimental.pallas import tpu_sc as plsc

_NC = 2
_NS = 16
_NW = _NC * _NS
_ECH = 80
_PCH = 128


def _node_pad(N):
    unit = _NS * 8
    return ((N + 1 + unit - 1) // unit) * unit


@functools.lru_cache(maxsize=None)
def _make_edge_agg(N, D, E_pad, with_deg):
    lanes = 16
    N_pad = _node_pad(N)
    RPT = N_pad // _NS
    EPW = E_pad // _NW
    NCH = EPW // _ECH
    assert E_pad % (_NW * _ECH) == 0 and NCH >= 16
    ZR = _ECH
    RPT16 = ((RPT + 15) // 16) * 16

    out_type = [jax.ShapeDtypeStruct((_NC, N_pad, D), jnp.float32)]
    scratch = []
    for _ in range(8):
        scratch += [
            pltpu.VMEM((2, _ECH), jnp.int32),
            pltpu.SemaphoreType.DMA,
        ]
    for _ in range(4):
        scratch += [
            pltpu.VMEM((_ECH, D), jnp.float32),
            pltpu.SemaphoreType.DMA,
            pltpu.SemaphoreType.DMA,
        ]
    scratch += [
        pltpu.VMEM_SHARED((N_pad, D), jnp.float32),
    ]
    if with_deg:
        out_type.append(jax.ShapeDtypeStruct((_NC * N_pad,), jnp.float32))
        scratch += [
            pltpu.VMEM((_ECH,), jnp.float32),
            pltpu.VMEM((RPT16,), jnp.float32),
            pltpu.VMEM_SHARED((N_pad,), jnp.float32),
        ]

    mesh = plsc.VectorSubcoreMesh(core_axis_name="c", subcore_axis_name="s")

    def body(h_hbm, combo_hbm, *refs):
        agg_out = refs[0]
        k = 1
        if with_deg:
            deg_out = refs[1]
            k = 2
        cbufs = [refs[k + 2 * i:k + 2 * (i + 1)] for i in range(8)]
        k += 16
        rbufs = [refs[k + 3 * i:k + 3 * (i + 1)] for i in range(4)]
        k += 12
        agg_sh = refs[k]
        if with_deg:
            ones_v, zdeg, deg_sh = refs[k + 1:k + 4]
        zbuf = rbufs[0][0]

        cid = lax.axis_index("c")
        sid = lax.axis_index("s")
        wid = cid * _NS + sid
        row0 = sid * RPT
        wbase = wid * NCH

        zero16 = jnp.zeros((16,), jnp.float32)

        def zrow(r, _):
            for j in range(D // lanes):
                zbuf[r, pl.ds(j * lanes, lanes)] = zero16
            return 0
        lax.fori_loop(0, ZR, zrow, 0)

        off = 0
        while off < RPT:
            sz = min(ZR, RPT - off)
            pltpu.sync_copy(zbuf.at[pl.ds(0, sz), :],
                            agg_sh.at[pl.ds(row0 + off, sz), :])
            off += sz

        if with_deg:
            one16 = jnp.ones((16,), jnp.float32)
            for j in range(_ECH // 16):
                ones_v[pl.ds(j * 16, 16)] = one16

            def zdrow(r, _):
                zdeg[pl.ds(r * 16, 16)] = zero16
                return 0
            lax.fori_loop(0, RPT16 // 16, zdrow, 0)
            pltpu.sync_copy(zdeg.at[pl.ds(0, RPT)],
                            deg_sh.at[pl.ds(row0, RPT)])

        plsc.subcore_barrier()

        def idx_start(c, ci):
            cb, isem = cbufs[ci]
            pltpu.async_copy(combo_hbm.at[wbase + c], cb, isem)

        def gather_start(c, ci, ri):
            cb, isem = cbufs[ci]
            rv, gs, _ = rbufs[ri]
            pltpu.make_async_copy(combo_hbm.at[wbase + c], cb, isem).wait()
            pltpu.async_copy(h_hbm.at[cb.at[0]], rv, gs)

        def scatter_issue(ci, ri):
            cb, _ = cbufs[ci]
            rv, gs, ss = rbufs[ri]
            pltpu.make_async_copy(h_hbm.at[cb.at[0]], rv, gs).wait()
            pltpu.async_copy(rv, agg_sh.at[cb.at[1]], ss, add=True)
            if with_deg:
                pltpu.async_copy(ones_v, deg_sh.at[cb.at[1]], ss, add=True)

        def scatter_wait(ci, ri):
            cb, _ = cbufs[ci]
            rv, _, ss = rbufs[ri]
            pltpu.make_async_copy(rv, agg_sh.at[cb.at[1]], ss).wait()
            if with_deg:
                pltpu.make_async_copy(ones_v, deg_sh.at[cb.at[1]], ss).wait()

        for c in range(4):
            idx_start(c, c)
        gather_start(0, 0, 0)
        gather_start(1, 1, 1)

        for c in range(8):
            pf = c + 2
            if c >= 2:
                scatter_wait((c - 2) % 8, (c - 2) % 4)
            gather_start(pf, pf % 8, pf % 4)
            idx_start(c + 4, (c + 4) % 8)
            scatter_issue(c % 8, c % 4)

        gt = (NCH - 4) // 8

        def grp(g, _):
            c0 = 8 * g
            for b in range(8):
                c = c0 + b
                scatter_wait((b - 2) % 8, (b - 2) % 4)
                gather_start(c + 2, (b + 2) % 8, (b + 2) % 4)
                idx_start(c + 4, (b + 4) % 8)
                scatter_issue(b % 8, b % 4)
            return 0
        if gt > 1:
            lax.fori_loop(1, gt, grp, 0)

        for c in range(8 * gt, NCH):
            if c >= 2:
                scatter_wait((c - 2) % 8, (c - 2) % 4)
            if c + 2 < NCH:
                gather_start(c + 2, (c + 2) % 8, (c + 2) % 4)
            if c + 4 < NCH:
                idx_start(c + 4, (c + 4) % 8)
            scatter_issue(c % 8, c % 4)
        scatter_wait((NCH - 2) % 8, (NCH - 2) % 4)
        scatter_wait((NCH - 1) % 8, (NCH - 1) % 4)

        plsc.subcore_barrier()

        off = 0
        while off < RPT:
            sz = min(ZR, RPT - off)
            pltpu.sync_copy(agg_sh.at[pl.ds(row0 + off, sz), :],
                            zbuf.at[pl.ds(0, sz), :])
            pltpu.sync_copy(zbuf.at[pl.ds(0, sz), :],
                            agg_out.at[cid, pl.ds(row0 + off, sz), :])
            off += sz
        if with_deg:
            pltpu.sync_copy(deg_sh.at[pl.ds(row0, RPT)], zdeg.at[pl.ds(0, RPT)])
            pltpu.sync_copy(zdeg.at[pl.ds(0, RPT)],
                            deg_out.at[pl.ds(cid * N_pad + row0, RPT)])

    return pl.kernel(body, out_type=tuple(out_type), mesh=mesh,
                     scratch_types=tuple(scratch))


@functools.lru_cache(maxsize=None)
def _make_pair_gather(N, D, PG):
    PPW = PG // _NW
    NCH = PPW // _PCH
    assert PG % (_NW * _PCH) == 0 and NCH % 2 == 1
    mesh = plsc.VectorSubcoreMesh(core_axis_name="c", subcore_axis_name="s")
    out_type = (jax.ShapeDtypeStruct((PG, D), jnp.float32),
                jax.ShapeDtypeStruct((PG, D), jnp.float32))
    scratch = []
    for _ in range(2):
        scratch += [
            pltpu.VMEM((2, _PCH), jnp.int32),
            pltpu.VMEM((_PCH, D), jnp.float32),
            pltpu.VMEM((_PCH, D), jnp.float32),
            pltpu.SemaphoreType.DMA,
        ]

    def body(h_hbm, combo_hbm, hs_out, hd_out, *refs):
        bufs = [refs[4 * i:4 * (i + 1)] for i in range(2)]
        cid = lax.axis_index("c")
        sid = lax.axis_index("s")
        wid = cid * _NS + sid
        wbase = wid * NCH

        def lg(c, i):
            cb, rs_v, rd_v, gs = bufs[i]
            pltpu.sync_copy(combo_hbm.at[wbase + c], cb)
            pltpu.async_copy(h_hbm.at[cb.at[0]], rs_v, gs)
            pltpu.async_copy(h_hbm.at[cb.at[1]], rd_v, gs)

        def finish(c, i):
            cb, rs_v, rd_v, gs = bufs[i]
            pltpu.make_async_copy(h_hbm.at[cb.at[0]], rs_v, gs).wait()
            pltpu.make_async_copy(h_hbm.at[cb.at[1]], rd_v, gs).wait()
            b = (wbase + c) * _PCH
            pltpu.sync_copy(rs_v, hs_out.at[pl.ds(b, _PCH), :])
            pltpu.sync_copy(rd_v, hd_out.at[pl.ds(b, _PCH), :])

        lg(0, 0)

        def pipe(jj, _):
            c = 2 * jj
            lg(c + 1, 1)
            finish(c, 0)
            lg(c + 2, 0)
            finish(c + 1, 1)
            return 0
        lax.fori_loop(0, (NCH - 1) // 2, pipe, 0)
        finish(NCH - 1, 0)

    return pl.kernel(body, out_type=out_type, mesh=mesh,
                     scratch_types=tuple(scratch))


def _bn_relu(z, g, b):
    m = jnp.mean(z, axis=0, keepdims=True)
    v = jnp.mean((z - m) ** 2, axis=0, keepdims=True)
    return jnp.maximum((z - m) / jnp.sqrt(v + 1e-5) * g + b, 0.0)


def _mlp_layer(h, aggp, degp, W1, g1, b1, W2, g2, b2, outer):
    N, D = h.shape
    H = W1.shape[1]

    def body(h_ref, agg_ref, deg_ref, W1_ref, g1_ref, b1_ref, W2_ref,
             g2_ref, b2_ref, *rest):
        if outer is not None:
            og_ref, ob_ref, out_ref = rest
        else:
            (out_ref,) = rest
        agg = agg_ref[0, :N, :] + agg_ref[1, :N, :]
        deg = deg_ref[0, :N, :] + deg_ref[1, :N, :]
        r = h_ref[...] + agg / jnp.maximum(deg, 1.0)
        z = jnp.dot(r, W1_ref[...], preferred_element_type=jnp.float32)
        z = _bn_relu(z, g1_ref[...], b1_ref[...])
        z = jnp.dot(z, W2_ref[...], preferred_element_type=jnp.float32)
        z = _bn_relu(z, g2_ref[...], b2_ref[...])
        if outer is not None:
            z = _bn_relu(z, og_ref[...], ob_ref[...])
        out_ref[...] = z

    args = [h, aggp, degp, W1, g1.reshape(1, -1), b1.reshape(1, -1),
            W2, g2.reshape(1, -1), b2.reshape(1, -1)]
    if outer is not None:
        og, ob = outer
        args += [og.reshape(1, -1), ob.reshape(1, -1)]
    return pl.pallas_call(
        body, out_shape=jax.ShapeDtypeStruct((N, H), jnp.float32))(*args)


def _predictor(hs, hd, M, P, W1, b1, W2, b2, W3, b3):
    def body(hs_ref, hd_ref, w1, b1r, w2, b2r, w3, b3r, out_ref):
        t = hs_ref[:M, :] * hd_ref[:M, :]
        t = jnp.maximum(
            jnp.dot(t, w1[...], preferred_element_type=jnp.float32) + b1r[...],
            0.0)
        t = jnp.maximum(
            jnp.dot(t, w2[...], preferred_element_type=jnp.float32) + b2r[...],
            0.0)
        t = jnp.dot(t, w3[...], preferred_element_type=jnp.float32) + b3r[...]
        out_ref[0] = t[:P]
        out_ref[1] = t[P:]

    return pl.pallas_call(
        body, out_shape=jax.ShapeDtypeStruct((2, P, 1), jnp.float32))(
            hs, hd, W1, b1.reshape(1, -1), W2, b2.reshape(1, -1),
            W3, b3.reshape(1, -1))


def _chunked(idx_a, idx_b, ch):
    return jnp.stack([idx_a.reshape(-1, ch), idx_b.reshape(-1, ch)], axis=1)


def kernel(x, edge_index, pos_edge_index, neg_edge_index, params):
    N, D = x.shape
    E = edge_index.shape[1]
    P = pos_edge_index.shape[1]
    layers = params['layers']
    outer_bn = params['outer_bn']
    pp = params['pred']
    L = len(layers)
    N_pad = _node_pad(N)

    assert E % _NW == 0
    epw_real = E // _NW
    nch = -(-epw_real // _ECH)
    epw = nch * _ECH
    wpad = epw - epw_real
    src = edge_index[0].reshape(_NW, epw_real)
    dst = edge_index[1].reshape(_NW, epw_real)
    if wpad:
        src = jnp.pad(src, ((0, 0), (0, wpad)))
        spill = (N + jnp.arange(wpad, dtype=jnp.int32) % (N_pad - N))
        dst = jnp.concatenate(
            [dst, jnp.broadcast_to(spill, (_NW, wpad)).astype(jnp.int32)],
            axis=1)
    E_pad = _NW * epw
    ecombo = _chunked(src.reshape(-1), dst.reshape(-1), _ECH)

    agg_deg = _make_edge_agg(N, D, E_pad, True)
    agg_only = _make_edge_agg(N, D, E_pad, False)

    h = x
    degp = None
    for l in range(L):
        p = layers[l]
        if l == 0:
            aggp, degp = agg_deg(h, ecombo)
            degp = degp.reshape(2, -1)[:, :, None]
        else:
            (aggp,) = agg_only(h, ecombo)
        outer = ((outer_bn[l]['g'], outer_bn[l]['b'])
                 if l != L - 1 else None)
        h = _mlp_layer(h, aggp, degp, p['W1'], p['bn1_g'], p['bn1_b'],
                       p['W2'], p['bn2_g'], p['bn2_b'], outer)

    M = 2 * P
    PG = ((M + _NW * _PCH - 1) // (_NW * _PCH)) * (_NW * _PCH)
    ppad = (jnp.arange(PG - M, dtype=jnp.int32) * 37) % N
    sidx = jnp.concatenate([pos_edge_index[0], neg_edge_index[0], ppad])
    didx = jnp.concatenate([pos_edge_index[1], neg_edge_index[1], ppad])
    hs, hd = _make_pair_gather(N, D, PG)(h, _chunked(sidx, didx, _PCH))

    return _predictor(hs, hd, M, P, pp['W1'], pp['b1'], pp['W2'], pp['b2'],
                      pp['W3'], pp['b3'])


# speedup vs baseline: 1.0467x; 1.0467x over previous
import functools

import jax
import jax.numpy as jnp
from jax import lax
from jax.experimental import pallas as pl
from jax.experimental.pallas import tpu as pltpu
from jax.experimental.pallas import tpu_sc as plsc

_NC = 2
_NS = 16
_NW = _NC * _NS
_ECH = 80
_PCH = 128


def _node_pad(N):
    unit = _NS * 8
    return ((N + 1 + unit - 1) // unit) * unit


@functools.lru_cache(maxsize=None)
def _make_edge_agg(N, D, E_pad, with_deg):
    lanes = 16
    N_pad = _node_pad(N)
    RPT = N_pad // _NS
    EPW = E_pad // _NW
    NCH = EPW // _ECH
    assert E_pad % (_NW * _ECH) == 0 and NCH >= 8
    ZR = _ECH
    RPT16 = ((RPT + 15) // 16) * 16

    out_type = [jax.ShapeDtypeStruct((_NC, N_pad, D), jnp.float32)]
    scratch = []
    for _ in range(4):
        scratch += [
            pltpu.VMEM((2, _ECH), jnp.int32),
            pltpu.VMEM((_ECH, D), jnp.float32),
            pltpu.SemaphoreType.DMA,
            pltpu.SemaphoreType.DMA,
        ]
    scratch += [
        pltpu.VMEM_SHARED((N_pad, D), jnp.float32),
    ]
    if with_deg:
        out_type.append(jax.ShapeDtypeStruct((_NC * N_pad,), jnp.float32))
        scratch += [
            pltpu.VMEM((_ECH,), jnp.float32),
            pltpu.VMEM((RPT16,), jnp.float32),
            pltpu.VMEM_SHARED((N_pad,), jnp.float32),
        ]

    mesh = plsc.VectorSubcoreMesh(core_axis_name="c", subcore_axis_name="s")

    def body(h_hbm, combo_hbm, *refs):
        agg_out = refs[0]
        k = 1
        if with_deg:
            deg_out = refs[1]
            k = 2
        bufs = [refs[k + 4 * i:k + 4 * (i + 1)] for i in range(4)]
        k += 16
        agg_sh = refs[k]
        if with_deg:
            ones_v, zdeg, deg_sh = refs[k + 1:k + 4]
        zbuf = bufs[0][1]

        cid = lax.axis_index("c")
        sid = lax.axis_index("s")
        wid = cid * _NS + sid
        row0 = sid * RPT
        wbase = wid * NCH

        zero16 = jnp.zeros((16,), jnp.float32)

        def zrow(r, _):
            for j in range(D // lanes):
                zbuf[r, pl.ds(j * lanes, lanes)] = zero16
            return 0
        lax.fori_loop(0, ZR, zrow, 0)

        off = 0
        while off < RPT:
            sz = min(ZR, RPT - off)
            pltpu.sync_copy(zbuf.at[pl.ds(0, sz), :],
                            agg_sh.at[pl.ds(row0 + off, sz), :])
            off += sz

        if with_deg:
            one16 = jnp.ones((16,), jnp.float32)
            for j in range(_ECH // 16):
                ones_v[pl.ds(j * 16, 16)] = one16

            def zdrow(r, _):
                zdeg[pl.ds(r * 16, 16)] = zero16
                return 0
            lax.fori_loop(0, RPT16 // 16, zdrow, 0)
            pltpu.sync_copy(zdeg.at[pl.ds(0, RPT)],
                            deg_sh.at[pl.ds(row0, RPT)])

        plsc.subcore_barrier()

        def lg(c, i):
            cb, rv, gs, _ = bufs[i]
            pltpu.sync_copy(combo_hbm.at[wbase + c], cb)
            pltpu.async_copy(h_hbm.at[cb.at[0]], rv, gs)

        def finish(i):
            cb, rv, gs, ss = bufs[i]
            pltpu.make_async_copy(h_hbm.at[cb.at[0]], rv, gs).wait()
            pltpu.async_copy(rv, agg_sh.at[cb.at[1]], ss, add=True)
            if with_deg:
                pltpu.async_copy(ones_v, deg_sh.at[cb.at[1]], ss, add=True)

        def wait_scatter(i):
            cb, rv, _, ss = bufs[i]
            pltpu.make_async_copy(rv, agg_sh.at[cb.at[1]], ss).wait()
            if with_deg:
                pltpu.make_async_copy(ones_v, deg_sh.at[cb.at[1]], ss).wait()

        def step_static(c):
            pf = c + 2
            if pf < NCH:
                if pf >= 4:
                    wait_scatter(pf % 4)
                lg(pf, pf % 4)
            finish(c % 4)

        lg(0, 0)
        lg(1, 1)
        for c in range(0, 4):
            step_static(c)

        gt = (NCH - 6) // 4 + 1

        def grp(g, _):
            c0 = 4 * g
            for b in range(4):
                i_pf = (b + 2) % 4
                wait_scatter(i_pf)
                lg(c0 + b + 2, i_pf)
                finish(b)
            return 0
        if gt > 1:
            lax.fori_loop(1, gt, grp, 0)
        for c in range(4 * gt, NCH):
            step_static(c)
        for i in range(4):
            wait_scatter(i)

        plsc.subcore_barrier()

        off = 0
        while off < RPT:
            sz = min(ZR, RPT - off)
            pltpu.sync_copy(agg_sh.at[pl.ds(row0 + off, sz), :],
                            zbuf.at[pl.ds(0, sz), :])
            pltpu.sync_copy(zbuf.at[pl.ds(0, sz), :],
                            agg_out.at[cid, pl.ds(row0 + off, sz), :])
            off += sz
        if with_deg:
            pltpu.sync_copy(deg_sh.at[pl.ds(row0, RPT)], zdeg.at[pl.ds(0, RPT)])
            pltpu.sync_copy(zdeg.at[pl.ds(0, RPT)],
                            deg_out.at[pl.ds(cid * N_pad + row0, RPT)])

    return pl.kernel(body, out_type=tuple(out_type), mesh=mesh,
                     scratch_types=tuple(scratch))


@functools.lru_cache(maxsize=None)
def _make_pair_gather(N, D, PG):
    PPW = PG // _NW
    NCH = PPW // _PCH
    assert PG % (_NW * _PCH) == 0 and NCH % 2 == 1
    mesh = plsc.VectorSubcoreMesh(core_axis_name="c", subcore_axis_name="s")
    out_type = (jax.ShapeDtypeStruct((PG, D), jnp.float32),
                jax.ShapeDtypeStruct((PG, D), jnp.float32))
    scratch = []
    for _ in range(2):
        scratch += [
            pltpu.VMEM((2, _PCH), jnp.int32),
            pltpu.VMEM((_PCH, D), jnp.float32),
            pltpu.VMEM((_PCH, D), jnp.float32),
            pltpu.SemaphoreType.DMA,
        ]

    def body(h_hbm, combo_hbm, hs_out, hd_out, *refs):
        bufs = [refs[4 * i:4 * (i + 1)] for i in range(2)]
        cid = lax.axis_index("c")
        sid = lax.axis_index("s")
        wid = cid * _NS + sid
        wbase = wid * NCH

        def lg(c, i):
            cb, rs_v, rd_v, gs = bufs[i]
            pltpu.sync_copy(combo_hbm.at[wbase + c], cb)
            pltpu.async_copy(h_hbm.at[cb.at[0]], rs_v, gs)
            pltpu.async_copy(h_hbm.at[cb.at[1]], rd_v, gs)

        def finish(c, i):
            cb, rs_v, rd_v, gs = bufs[i]
            pltpu.make_async_copy(h_hbm.at[cb.at[0]], rs_v, gs).wait()
            pltpu.make_async_copy(h_hbm.at[cb.at[1]], rd_v, gs).wait()
            b = (wbase + c) * _PCH
            pltpu.sync_copy(rs_v, hs_out.at[pl.ds(b, _PCH), :])
            pltpu.sync_copy(rd_v, hd_out.at[pl.ds(b, _PCH), :])

        lg(0, 0)

        def pipe(jj, _):
            c = 2 * jj
            lg(c + 1, 1)
            finish(c, 0)
            lg(c + 2, 0)
            finish(c + 1, 1)
            return 0
        lax.fori_loop(0, (NCH - 1) // 2, pipe, 0)
        finish(NCH - 1, 0)

    return pl.kernel(body, out_type=out_type, mesh=mesh,
                     scratch_types=tuple(scratch))


def _bn_relu(z, g, b):
    m = jnp.mean(z, axis=0, keepdims=True)
    v = jnp.mean((z - m) ** 2, axis=0, keepdims=True)
    return jnp.maximum((z - m) / jnp.sqrt(v + 1e-5) * g + b, 0.0)


def _mlp_layer(h, aggp, degp, W1, g1, b1, W2, g2, b2, outer):
    N, D = h.shape
    H = W1.shape[1]

    def body(h_ref, agg_ref, deg_ref, W1_ref, g1_ref, b1_ref, W2_ref,
             g2_ref, b2_ref, *rest):
        if outer is not None:
            og_ref, ob_ref, out_ref = rest
        else:
            (out_ref,) = rest
        agg = agg_ref[0, :N, :] + agg_ref[1, :N, :]
        deg = deg_ref[0:1, :N] + deg_ref[1:2, :N]
        deg_col = jnp.swapaxes(deg, 0, 1)
        r = h_ref[...] + agg / jnp.maximum(deg_col, 1.0)
        z = jnp.dot(r, W1_ref[...], preferred_element_type=jnp.float32)
        z = _bn_relu(z, g1_ref[...], b1_ref[...])
        z = jnp.dot(z, W2_ref[...], preferred_element_type=jnp.float32)
        z = _bn_relu(z, g2_ref[...], b2_ref[...])
        if outer is not None:
            z = _bn_relu(z, og_ref[...], ob_ref[...])
        out_ref[...] = z

    args = [h, aggp, degp, W1, g1.reshape(1, -1), b1.reshape(1, -1),
            W2, g2.reshape(1, -1), b2.reshape(1, -1)]
    if outer is not None:
        og, ob = outer
        args += [og.reshape(1, -1), ob.reshape(1, -1)]
    return pl.pallas_call(
        body, out_shape=jax.ShapeDtypeStruct((N, H), jnp.float32))(*args)


def _predictor(hs, hd, M, P, W1, b1, W2, b2, W3, b3):
    def body(hs_ref, hd_ref, w1, b1r, w2, b2r, w3, b3r, out_ref):
        t = hs_ref[:M, :] * hd_ref[:M, :]
        t = jnp.maximum(
            jnp.dot(t, w1[...], preferred_element_type=jnp.float32) + b1r[...],
            0.0)
        t = jnp.maximum(
            jnp.dot(t, w2[...], preferred_element_type=jnp.float32) + b2r[...],
            0.0)
        t = jnp.dot(t, w3[...], preferred_element_type=jnp.float32) + b3r[...]
        out_ref[0] = t[:P]
        out_ref[1] = t[P:]

    return pl.pallas_call(
        body, out_shape=jax.ShapeDtypeStruct((2, P, 1), jnp.float32))(
            hs, hd, W1, b1.reshape(1, -1), W2, b2.reshape(1, -1),
            W3, b3.reshape(1, -1))


def _chunked(idx_a, idx_b, ch):
    return jnp.stack([idx_a.reshape(-1, ch), idx_b.reshape(-1, ch)], axis=1)


def kernel(x, edge_index, pos_edge_index, neg_edge_index, params):
    N, D = x.shape
    E = edge_index.shape[1]
    P = pos_edge_index.shape[1]
    layers = params['layers']
    outer_bn = params['outer_bn']
    pp = params['pred']
    L = len(layers)
    N_pad = _node_pad(N)

    assert E % _NW == 0
    epw_real = E // _NW
    nch = -(-epw_real // _ECH)
    epw = nch * _ECH
    wpad = epw - epw_real
    src = edge_index[0].reshape(_NW, epw_real)
    dst = edge_index[1].reshape(_NW, epw_real)
    if wpad:
        src = jnp.pad(src, ((0, 0), (0, wpad)))
        spill = (N + jnp.arange(wpad, dtype=jnp.int32) % (N_pad - N))
        dst = jnp.concatenate(
            [dst, jnp.broadcast_to(spill, (_NW, wpad)).astype(jnp.int32)],
            axis=1)
    E_pad = _NW * epw
    ecombo = _chunked(src.reshape(-1), dst.reshape(-1), _ECH)

    agg_deg = _make_edge_agg(N, D, E_pad, True)
    agg_only = _make_edge_agg(N, D, E_pad, False)

    h = x
    degp = None
    for l in range(L):
        p = layers[l]
        if l == 0:
            aggp, degp = agg_deg(h, ecombo)
            degp = degp.reshape(2, -1)
        else:
            (aggp,) = agg_only(h, ecombo)
        outer = ((outer_bn[l]['g'], outer_bn[l]['b'])
                 if l != L - 1 else None)
        h = _mlp_layer(h, aggp, degp, p['W1'], p['bn1_g'], p['bn1_b'],
                       p['W2'], p['bn2_g'], p['bn2_b'], outer)

    M = 2 * P
    PG = ((M + _NW * _PCH - 1) // (_NW * _PCH)) * (_NW * _PCH)
    ppad = (jnp.arange(PG - M, dtype=jnp.int32) * 37) % N
    sidx = jnp.concatenate([pos_edge_index[0], neg_edge_index[0], ppad])
    didx = jnp.concatenate([pos_edge_index[1], neg_edge_index[1], ppad])
    hs, hd = _make_pair_gather(N, D, PG)(h, _chunked(sidx, didx, _PCH))

    return _predictor(hs, hd, M, P, pp['W1'], pp['b1'], pp['W2'], pp['b2'],
                      pp['W3'], pp['b3'])
